# baseline (speedup 1.0000x reference)
import functools

import jax
import jax.numpy as jnp
from jax import lax
from jax.experimental import pallas as pl
from jax.experimental.pallas import tpu as pltpu
from jax.experimental.pallas import tpu_sc as plsc

NN = 10000
FD = 128
H2_ = 64
NE = 320000
NC_, NS_ = 2, 16
NW_ = NC_ * NS_
CH = 128
NP_ = 10240
DPT = NP_ // NS_
RPT = 632
RPT_LAST = NN - 15 * RPT
GRP = 6
NCH = 84
EPW2 = NCH * CH
NE2 = NW_ * EPW2
NSINK = 64
EPW = NE // NW_
NFULL = EPW // CH
REM = EPW - NFULL * CH
RING_D = 4

_mesh = plsc.VectorSubcoreMesh(core_axis_name="c", subcore_axis_name="s")


DEG_RING = 8


@functools.partial(
    pl.kernel,
    out_type=jax.ShapeDtypeStruct((NC_, NP_), jnp.float32),
    mesh=_mesh,
    scratch_types=[
        pltpu.VMEM((DEG_RING, CH), jnp.int32),
        pltpu.VMEM((CH,), jnp.float32),
        pltpu.VMEM((REM,), jnp.int32),
        pltpu.VMEM((REM,), jnp.float32),
        pltpu.VMEM((DPT,), jnp.float32),
        [pltpu.SemaphoreType.DMA for _ in range(DEG_RING)],
        [pltpu.SemaphoreType.DMA for _ in range(DEG_RING)],
        pltpu.VMEM_SHARED((NP_,), jnp.float32),
    ],
    compiler_params=pltpu.CompilerParams(use_tc_tiling_on_sc=False),
)
def _deg_sc(ei_hbm, out_hbm, dstb, ones_v, dstr, onesr, zbuf_v,
            dsems, ssems, deg_sh):
    c = lax.axis_index("c")
    s = lax.axis_index("s")
    wid = s * NC_ + c
    soff = pl.multiple_of(wid * EPW, 8)
    dof = pl.multiple_of(s * DPT, 8)

    def _load(j, p):
        off = pl.multiple_of(soff + j * CH, 8)
        pltpu.async_copy(ei_hbm.at[1, pl.ds(off, CH)], dstb.at[p], dsems[p])

    def _wload(j, p):
        off = pl.multiple_of(soff + j * CH, 8)
        pltpu.make_async_copy(ei_hbm.at[1, pl.ds(off, CH)], dstb.at[p],
                              dsems[p]).wait()

    def _scat(j, p):
        pltpu.async_copy(ones_v, deg_sh.at[dstb.at[p]], ssems[p], add=True)

    def _wscat(j, p):
        pltpu.make_async_copy(ones_v, deg_sh.at[dstb.at[p]], ssems[p]).wait()

    for q in range(4):
        _load(q, q)
    for i in range(DPT // 16):
        zbuf_v[pl.ds(i * 16, 16)] = jnp.zeros((16,), jnp.float32)
    for i in range(CH // 16):
        ones_v[pl.ds(i * 16, 16)] = jnp.full((16,), 1.0, jnp.float32)
    onesr_vals = jnp.full((16,), 1.0, jnp.float32)
    onesr[pl.ds(0, 16)] = onesr_vals
    pltpu.sync_copy(zbuf_v, deg_sh.at[pl.ds(dof, DPT)])
    plsc.subcore_barrier()

    def istep(j, p, do_wscat, do_load):
        _wload(j, p)
        _scat(j, p)
        if do_wscat:
            _wscat(j - 4, (p - 4) % DEG_RING)
        if do_load:
            _load(j + 4, (p + 4) % DEG_RING)

    for p in range(DEG_RING):
        istep(p, p, p >= 4, True)

    def dgrp(t, carry):
        for p in range(DEG_RING):
            istep(t * DEG_RING + p, p, True, True)
        return carry

    lax.fori_loop(1, NFULL // DEG_RING, dgrp, 0)
    for p in range(NFULL % DEG_RING):
        j = (NFULL // DEG_RING) * DEG_RING + p
        istep(j, p, True, j + 4 < NFULL)
    for k in range(NFULL - 4, NFULL):
        _wscat(k, k % DEG_RING)

    offr = pl.multiple_of(soff + NFULL * CH, 8)
    pltpu.sync_copy(ei_hbm.at[1, pl.ds(offr, REM)], dstr)
    pltpu.sync_copy(onesr, deg_sh.at[dstr], add=True)
    plsc.subcore_barrier()
    pltpu.sync_copy(deg_sh.at[pl.ds(dof, DPT)],
                    out_hbm.at[c, pl.ds(dof, DPT)])


def _make_agg(F, RR):
    L = RR - 1
    IR = 6

    @functools.partial(
        pl.kernel,
        out_type=jax.ShapeDtypeStruct((NC_, NN, F), jnp.float32),
        mesh=_mesh,
        scratch_types=[
            pltpu.VMEM((IR, CH), jnp.int32),
            pltpu.VMEM((IR, CH), jnp.int32),
            [pltpu.VMEM((CH, F), jnp.float32) for _ in range(RR)],
            pltpu.VMEM((REM,), jnp.int32),
            pltpu.VMEM((REM,), jnp.int32),
            [pltpu.SemaphoreType.DMA for _ in range(RR)],
            [pltpu.SemaphoreType.DMA for _ in range(IR)],
            [pltpu.SemaphoreType.DMA for _ in range(IR)],
            pltpu.VMEM_SHARED((NN, F), jnp.float32),
        ],
        compiler_params=pltpu.CompilerParams(use_tc_tiling_on_sc=False),
    )
    def agg(g_hbm, ei_hbm, out_hbm,
            srcb, dstb, rows, srcr, dstr,
            gsems, isems, dsems, acc_sh):
        c = lax.axis_index("c")
        s = lax.axis_index("s")
        wid = s * NC_ + c
        row0 = pl.multiple_of(s * RPT, 8)
        soff = pl.multiple_of(wid * EPW, 8)

        @pl.when(c == 0)
        def _():
            @pl.when(s < 15)
            def _():
                pltpu.sync_copy(g_hbm.at[pl.ds(row0, RPT), :],
                                acc_sh.at[pl.ds(row0, RPT), :])

            @pl.when(s == 15)
            def _():
                pltpu.sync_copy(g_hbm.at[pl.ds(15 * RPT, RPT_LAST), :],
                                acc_sh.at[pl.ds(15 * RPT, RPT_LAST), :])

        @pl.when(c == 1)
        def _():
            def zrow(r, carry):
                for k in range(F // 16):
                    rows[0][r, pl.ds(k * 16, 16)] = jnp.zeros(
                        (16,), jnp.float32)
                return carry

            lax.fori_loop(0, CH, zrow, 0)

            @pl.when(s < 15)
            def _():
                for k in range(4):
                    pltpu.sync_copy(
                        rows[0].at[pl.ds(0, CH), :],
                        acc_sh.at[pl.ds(row0 + CH * k, CH), :])
                pltpu.sync_copy(
                    rows[0].at[pl.ds(0, RPT - 4 * CH), :],
                    acc_sh.at[pl.ds(row0 + 4 * CH, RPT - 4 * CH), :])

            @pl.when(s == 15)
            def _():
                for k in range(4):
                    pltpu.sync_copy(
                        rows[0].at[pl.ds(0, CH), :],
                        acc_sh.at[pl.ds(15 * RPT + CH * k, CH), :])
                pltpu.sync_copy(
                    rows[0].at[pl.ds(0, RPT_LAST - 4 * CH), :],
                    acc_sh.at[pl.ds(15 * RPT + 4 * CH, RPT_LAST - 4 * CH), :])

        plsc.subcore_barrier()

        def _src_load(j, q):
            off = pl.multiple_of(soff + j * CH, 8)
            pltpu.async_copy(ei_hbm.at[0, pl.ds(off, CH)], srcb.at[q],
                             isems[q])

        def _wait_src(j, q):
            off = pl.multiple_of(soff + j * CH, 8)
            pltpu.make_async_copy(ei_hbm.at[0, pl.ds(off, CH)], srcb.at[q],
                                  isems[q]).wait()

        def _dst_load(j, q):
            off = pl.multiple_of(soff + j * CH, 8)
            pltpu.async_copy(ei_hbm.at[1, pl.ds(off, CH)], dstb.at[q],
                             dsems[q])

        def _wait_dst(j, q):
            off = pl.multiple_of(soff + j * CH, 8)
            pltpu.make_async_copy(ei_hbm.at[1, pl.ds(off, CH)], dstb.at[q],
                                  dsems[q]).wait()

        def _gather(j, qi, qr):
            pltpu.async_copy(g_hbm.at[srcb.at[qi]], rows[qr], gsems[qr])

        def _wait_gather(j, qi, qr):
            pltpu.make_async_copy(g_hbm.at[srcb.at[qi]], rows[qr],
                                  gsems[qr]).wait()

        def step(j, p, do_loads, do_gather):
            _wait_gather(j, p % IR, p % RR)
            _wait_dst(j, p % IR)
            pltpu.sync_copy(rows[p % RR], acc_sh.at[dstb.at[p % IR]],
                            add=True)
            if do_loads:
                _dst_load(j + IR, p % IR)
                _src_load(j + IR, p % IR)
            if do_gather:
                _wait_src(j + L, (p + L) % IR)
                _gather(j + L, (p + L) % IR, (p + L) % RR)

        for q in range(IR):
            _src_load(q, q)
            _dst_load(q, q)
        for q in range(L):
            _wait_src(q, q)
            _gather(q, q, q % RR)

        def grp(t, carry):
            for p in range(IR):
                step(t * IR + p, p, True, True)
            return carry

        lax.fori_loop(0, NFULL // IR - 1, grp, 0)
        tail0 = (NFULL // IR - 1) * IR
        for p in range(NFULL - tail0):
            j = tail0 + p
            step(j, p, j + IR < NFULL, j + L < NFULL)

        offr = pl.multiple_of(soff + NFULL * CH, 8)
        pltpu.sync_copy(ei_hbm.at[0, pl.ds(offr, REM)], srcr)
        pltpu.sync_copy(ei_hbm.at[1, pl.ds(offr, REM)], dstr)
        pltpu.async_copy(g_hbm.at[srcr], rows[0].at[pl.ds(0, REM), :],
                         gsems[0]).wait()
        pltpu.sync_copy(rows[0].at[pl.ds(0, REM), :], acc_sh.at[dstr],
                        add=True)

        plsc.subcore_barrier()

        @pl.when(s < 15)
        def _():
            pltpu.sync_copy(acc_sh.at[pl.ds(row0, RPT), :],
                            out_hbm.at[c, pl.ds(row0, RPT), :])

        @pl.when(s == 15)
        def _():
            pltpu.sync_copy(acc_sh.at[pl.ds(15 * RPT, RPT_LAST), :],
                            out_hbm.at[c, pl.ds(15 * RPT, RPT_LAST), :])

    return agg


_agg128 = _make_agg(FD, 3)
_agg64 = _make_agg(H2_, 6)


_BR = 2000
_GRID = NN // _BR


def _t1a_body(x_ref, w1_ref, mm_ref):
    mm_ref[...] = jnp.dot(x_ref[...], w1_ref[...],
                          preferred_element_type=jnp.float32)


def _t1a(x, W1):
    return pl.pallas_call(
        _t1a_body,
        grid=(_GRID,),
        in_specs=[
            pl.BlockSpec((_BR, FD), lambda j: (j, 0)),
            pl.BlockSpec((FD, FD), lambda j: (0, 0)),
        ],
        out_specs=pl.BlockSpec((_BR, FD), lambda j: (j, 0)),
        out_shape=jax.ShapeDtypeStruct((NN, FD), jnp.float32),
    )(x, W1)


def _t1b_body(degT_ref, mm_ref, g_ref, dis_ref):
    deg = jnp.sum(degT_ref[...], axis=1, keepdims=True) + 1.0
    dis = lax.rsqrt(deg)
    g_ref[...] = mm_ref[...] * dis
    dis_ref[...] = dis


def _t1b(degT, mm):
    return pl.pallas_call(
        _t1b_body,
        grid=(_GRID,),
        in_specs=[
            pl.BlockSpec((_BR, 2), lambda j: (j, 0)),
            pl.BlockSpec((_BR, FD), lambda j: (j, 0)),
        ],
        out_specs=[
            pl.BlockSpec((_BR, FD), lambda j: (j, 0)),
            pl.BlockSpec((_BR, 1), lambda j: (j, 0)),
        ],
        out_shape=[
            jax.ShapeDtypeStruct((NN, FD), jnp.float32),
            jax.ShapeDtypeStruct((NN, 1), jnp.float32),
        ],
    )(degT, mm)


def _t2_body(p_ref, dis_ref, b1_ref, w2_ref, g2_ref):
    dis = dis_ref[...]
    h1 = jnp.maximum(dis * (p_ref[0] + p_ref[1]) + b1_ref[...], 0.0)
    g2_ref[...] = jnp.dot(h1, w2_ref[...],
                          preferred_element_type=jnp.float32) * dis


def _t2(p, dis, b1, W2):
    return pl.pallas_call(
        _t2_body,
        grid=(_GRID,),
        in_specs=[
            pl.BlockSpec((NC_, _BR, FD), lambda j: (0, j, 0)),
            pl.BlockSpec((_BR, 1), lambda j: (j, 0)),
            pl.BlockSpec((1, FD), lambda j: (0, 0)),
            pl.BlockSpec((FD, H2_), lambda j: (0, 0)),
        ],
        out_specs=pl.BlockSpec((_BR, H2_), lambda j: (j, 0)),
        out_shape=jax.ShapeDtypeStruct((NN, H2_), jnp.float32),
    )(p, dis, b1, W2)


def _t3_body(q_ref, dis_ref, b2_ref, w0_ref, w1_ref, bo_ref, o0_ref):
    dis = dis_ref[...]
    h2 = jnp.maximum(dis * (q_ref[0] + q_ref[1]) + b2_ref[...], 0.0)
    l0 = jnp.sum(h2 * w0_ref[...], axis=1, keepdims=True) + bo_ref[:, 0:1]
    l1 = jnp.sum(h2 * w1_ref[...], axis=1, keepdims=True) + bo_ref[:, 1:2]
    m = jnp.maximum(l0, l1)
    lse = m + jnp.log(jnp.exp(l0 - m) + jnp.exp(l1 - m))
    o0_ref[...] = jnp.concatenate([l0 - lse, l1 - lse], axis=1)


def _t3(q, dis, b2, w0, w1, bo2):
    return pl.pallas_call(
        _t3_body,
        grid=(_GRID,),
        in_specs=[
            pl.BlockSpec((NC_, _BR, H2_), lambda j: (0, j, 0)),
            pl.BlockSpec((_BR, 1), lambda j: (j, 0)),
            pl.BlockSpec((1, H2_), lambda j: (0, 0)),
            pl.BlockSpec((1, H2_), lambda j: (0, 0)),
            pl.BlockSpec((1, H2_), lambda j: (0, 0)),
            pl.BlockSpec((1, 2), lambda j: (0, 0)),
        ],
        out_specs=pl.BlockSpec((_BR, 2), lambda j: (j, 0)),
        out_shape=jax.ShapeDtypeStruct((NN, 2), jnp.float32),
    )(q, dis, b2, w0, w1, bo2)


def kernel(x, edge_index, W1, b1, W2, b2, Wo, bo):
    mm = _t1a(x, W1)
    degp = _deg_sc(edge_index)
    degT = jnp.transpose(degp[:, :NN])
    g1, dis = _t1b(degT, mm)
    p = _agg128(g1, edge_index)
    g2 = _t2(p, dis, b1.reshape(1, FD), W2)
    q = _agg64(g2, edge_index)
    return _t3(q, dis, b2.reshape(1, H2_),
               Wo[:, 0].reshape(1, H2_), Wo[:, 1].reshape(1, H2_),
               bo.reshape(1, 2))


# speedup vs baseline: 1.0048x; 1.0048x over previous
import functools

import jax
import jax.numpy as jnp
from jax import lax
from jax.experimental import pallas as pl
from jax.experimental.pallas import tpu as pltpu
from jax.experimental.pallas import tpu_sc as plsc

NN = 10000
FD = 128
H2_ = 64
NE = 320000
NC_, NS_ = 2, 16
NW_ = NC_ * NS_
CH = 128
NP_ = 10240
DPT = NP_ // NS_
RPT = 632
RPT_LAST = NN - 15 * RPT
GRP = 6
NCH = 84
EPW2 = NCH * CH
NE2 = NW_ * EPW2
NSINK = 64
EPW = NE // NW_
NFULL = EPW // CH
REM = EPW - NFULL * CH
RING_D = 4

_mesh = plsc.VectorSubcoreMesh(core_axis_name="c", subcore_axis_name="s")


DEG_RING = 8


@functools.partial(
    pl.kernel,
    out_type=jax.ShapeDtypeStruct((NC_, NP_), jnp.float32),
    mesh=_mesh,
    scratch_types=[
        pltpu.VMEM((DEG_RING, CH), jnp.int32),
        pltpu.VMEM((CH,), jnp.float32),
        pltpu.VMEM((REM,), jnp.int32),
        pltpu.VMEM((REM,), jnp.float32),
        pltpu.VMEM((DPT,), jnp.float32),
        [pltpu.SemaphoreType.DMA for _ in range(DEG_RING)],
        [pltpu.SemaphoreType.DMA for _ in range(DEG_RING)],
        pltpu.VMEM_SHARED((NP_,), jnp.float32),
    ],
    compiler_params=pltpu.CompilerParams(use_tc_tiling_on_sc=False),
)
def _deg_sc(ei_hbm, out_hbm, dstb, ones_v, dstr, onesr, zbuf_v,
            dsems, ssems, deg_sh):
    c = lax.axis_index("c")
    s = lax.axis_index("s")
    wid = s * NC_ + c
    soff = pl.multiple_of(wid * EPW, 8)
    dof = pl.multiple_of(s * DPT, 8)

    def _load(j, p):
        off = pl.multiple_of(soff + j * CH, 8)
        pltpu.async_copy(ei_hbm.at[1, pl.ds(off, CH)], dstb.at[p], dsems[p])

    def _wload(j, p):
        off = pl.multiple_of(soff + j * CH, 8)
        pltpu.make_async_copy(ei_hbm.at[1, pl.ds(off, CH)], dstb.at[p],
                              dsems[p]).wait()

    def _scat(j, p):
        pltpu.async_copy(ones_v, deg_sh.at[dstb.at[p]], ssems[p], add=True)

    def _wscat(j, p):
        pltpu.make_async_copy(ones_v, deg_sh.at[dstb.at[p]], ssems[p]).wait()

    for q in range(4):
        _load(q, q)
    for i in range(DPT // 16):
        zbuf_v[pl.ds(i * 16, 16)] = jnp.zeros((16,), jnp.float32)
    for i in range(CH // 16):
        ones_v[pl.ds(i * 16, 16)] = jnp.full((16,), 1.0, jnp.float32)
    onesr_vals = jnp.full((16,), 1.0, jnp.float32)
    onesr[pl.ds(0, 16)] = onesr_vals
    pltpu.sync_copy(zbuf_v, deg_sh.at[pl.ds(dof, DPT)])
    plsc.subcore_barrier()

    def istep(j, p, do_wscat, do_load):
        _wload(j, p)
        _scat(j, p)
        if do_wscat:
            _wscat(j - 4, (p - 4) % DEG_RING)
        if do_load:
            _load(j + 4, (p + 4) % DEG_RING)

    for p in range(DEG_RING):
        istep(p, p, p >= 4, True)

    def dgrp(t, carry):
        for p in range(DEG_RING):
            istep(t * DEG_RING + p, p, True, True)
        return carry

    lax.fori_loop(1, NFULL // DEG_RING, dgrp, 0)
    for p in range(NFULL % DEG_RING):
        j = (NFULL // DEG_RING) * DEG_RING + p
        istep(j, p, True, j + 4 < NFULL)
    for k in range(NFULL - 4, NFULL):
        _wscat(k, k % DEG_RING)

    offr = pl.multiple_of(soff + NFULL * CH, 8)
    pltpu.sync_copy(ei_hbm.at[1, pl.ds(offr, REM)], dstr)
    pltpu.sync_copy(onesr, deg_sh.at[dstr], add=True)
    plsc.subcore_barrier()
    pltpu.sync_copy(deg_sh.at[pl.ds(dof, DPT)],
                    out_hbm.at[c, pl.ds(dof, DPT)])


def _make_agg(F, RR):
    @functools.partial(
        pl.kernel,
        out_type=jax.ShapeDtypeStruct((NC_, NN, F), jnp.float32),
        mesh=_mesh,
        scratch_types=[
            pltpu.VMEM((EPW,), jnp.int32),
            pltpu.VMEM((RING_D, CH), jnp.int32),
            [pltpu.VMEM((CH, F), jnp.float32) for _ in range(RR)],
            pltpu.VMEM((REM,), jnp.int32),
            pltpu.VMEM((REM,), jnp.int32),
            pltpu.VMEM((REM, F), jnp.float32),
            [pltpu.SemaphoreType.DMA for _ in range(RR)],
            [pltpu.SemaphoreType.DMA for _ in range(RING_D)],
            pltpu.VMEM_SHARED((NN, F), jnp.float32),
        ],
        compiler_params=pltpu.CompilerParams(use_tc_tiling_on_sc=False),
    )
    def agg(g_hbm, ei_hbm, out_hbm,
            src_all, dstb, rows, srcr, dstr, rowsr,
            gsems, dsems, acc_sh):
        c = lax.axis_index("c")
        s = lax.axis_index("s")
        wid = s * NC_ + c
        row0 = pl.multiple_of(s * RPT, 8)
        soff = pl.multiple_of(wid * EPW, 8)

        @pl.when(c == 0)
        def _():
            @pl.when(s < 15)
            def _():
                pltpu.sync_copy(g_hbm.at[pl.ds(row0, RPT), :],
                                acc_sh.at[pl.ds(row0, RPT), :])

            @pl.when(s == 15)
            def _():
                pltpu.sync_copy(g_hbm.at[pl.ds(15 * RPT, RPT_LAST), :],
                                acc_sh.at[pl.ds(15 * RPT, RPT_LAST), :])

        @pl.when(c == 1)
        def _():
            def zrow(r, carry):
                for k in range(F // 16):
                    rows[0][r, pl.ds(k * 16, 16)] = jnp.zeros(
                        (16,), jnp.float32)
                return carry

            lax.fori_loop(0, CH, zrow, 0)

            @pl.when(s < 15)
            def _():
                for k in range(4):
                    pltpu.sync_copy(
                        rows[0].at[pl.ds(0, CH), :],
                        acc_sh.at[pl.ds(row0 + CH * k, CH), :])
                pltpu.sync_copy(
                    rows[0].at[pl.ds(0, RPT - 4 * CH), :],
                    acc_sh.at[pl.ds(row0 + 4 * CH, RPT - 4 * CH), :])

            @pl.when(s == 15)
            def _():
                for k in range(4):
                    pltpu.sync_copy(
                        rows[0].at[pl.ds(0, CH), :],
                        acc_sh.at[pl.ds(15 * RPT + CH * k, CH), :])
                pltpu.sync_copy(
                    rows[0].at[pl.ds(0, RPT_LAST - 4 * CH), :],
                    acc_sh.at[pl.ds(15 * RPT + 4 * CH, RPT_LAST - 4 * CH), :])

        plsc.subcore_barrier()

        pltpu.sync_copy(ei_hbm.at[0, pl.ds(soff, EPW)], src_all)

        def _gather(j, pr):
            ioff = pl.multiple_of(j * CH, 8)
            pltpu.async_copy(
                g_hbm.at[src_all.at[pl.ds(ioff, CH)]], rows[pr], gsems[pr])

        def _wait_gather(j, pr):
            ioff = pl.multiple_of(j * CH, 8)
            pltpu.make_async_copy(
                g_hbm.at[src_all.at[pl.ds(ioff, CH)]], rows[pr],
                gsems[pr]).wait()

        def _dst_load(j, pd):
            doff = pl.multiple_of(soff + j * CH, 8)
            pltpu.async_copy(ei_hbm.at[1, pl.ds(doff, CH)],
                             dstb.at[pd], dsems[pd])

        def _wait_dst(j, pd):
            doff = pl.multiple_of(soff + j * CH, 8)
            pltpu.make_async_copy(ei_hbm.at[1, pl.ds(doff, CH)],
                                  dstb.at[pd], dsems[pd]).wait()

        def step(j, p, do_dst, do_gather):
            pr = p % RR
            pd = p % RING_D
            _wait_gather(j, pr)
            _wait_dst(j, pd)
            pltpu.sync_copy(rows[pr], acc_sh.at[dstb.at[pd]], add=True)
            if do_dst:
                _dst_load(j + RING_D, pd)
            if do_gather:
                _gather(j + RR, pr)

        for q in range(RING_D):
            _dst_load(q, q)
        for q in range(RR):
            _gather(q, q % RR)

        def grp(t, carry):
            for p in range(RING_D):
                j = t * RING_D + p
                step(j, p, True, True)
            return carry

        lax.fori_loop(0, NFULL // RING_D - 1, grp, 0)

        tail0 = (NFULL // RING_D - 1) * RING_D
        for p in range(NFULL - tail0):
            j = tail0 + p
            step(j, p, j + RING_D < NFULL, j + RR < NFULL)

        offr = pl.multiple_of(soff + NFULL * CH, 8)
        pltpu.sync_copy(ei_hbm.at[0, pl.ds(offr, REM)], srcr)
        pltpu.sync_copy(ei_hbm.at[1, pl.ds(offr, REM)], dstr)
        pltpu.async_copy(g_hbm.at[srcr], rowsr, gsems[0]).wait()
        pltpu.sync_copy(rowsr, acc_sh.at[dstr], add=True)

        plsc.subcore_barrier()

        @pl.when(s < 15)
        def _():
            pltpu.sync_copy(acc_sh.at[pl.ds(row0, RPT), :],
                            out_hbm.at[c, pl.ds(row0, RPT), :])

        @pl.when(s == 15)
        def _():
            pltpu.sync_copy(acc_sh.at[pl.ds(15 * RPT, RPT_LAST), :],
                            out_hbm.at[c, pl.ds(15 * RPT, RPT_LAST), :])

    return agg


_agg128 = _make_agg(FD, 2)
_agg64 = _make_agg(H2_, 4)


_BR = 2000
_GRID = NN // _BR


def _t1a_body(x_ref, w1_ref, mm_ref):
    mm_ref[...] = jnp.dot(x_ref[...], w1_ref[...],
                          preferred_element_type=jnp.float32)


def _t1a(x, W1):
    return pl.pallas_call(
        _t1a_body,
        grid=(_GRID,),
        in_specs=[
            pl.BlockSpec((_BR, FD), lambda j: (j, 0)),
            pl.BlockSpec((FD, FD), lambda j: (0, 0)),
        ],
        out_specs=pl.BlockSpec((_BR, FD), lambda j: (j, 0)),
        out_shape=jax.ShapeDtypeStruct((NN, FD), jnp.float32),
    )(x, W1)


def _t1b_body(degT_ref, mm_ref, g_ref, dis_ref):
    deg = jnp.sum(degT_ref[...], axis=1, keepdims=True) + 1.0
    dis = lax.rsqrt(deg)
    g_ref[...] = mm_ref[...] * dis
    dis_ref[...] = dis


def _t1b(degT, mm):
    return pl.pallas_call(
        _t1b_body,
        grid=(_GRID,),
        in_specs=[
            pl.BlockSpec((_BR, 2), lambda j: (j, 0)),
            pl.BlockSpec((_BR, FD), lambda j: (j, 0)),
        ],
        out_specs=[
            pl.BlockSpec((_BR, FD), lambda j: (j, 0)),
            pl.BlockSpec((_BR, 1), lambda j: (j, 0)),
        ],
        out_shape=[
            jax.ShapeDtypeStruct((NN, FD), jnp.float32),
            jax.ShapeDtypeStruct((NN, 1), jnp.float32),
        ],
    )(degT, mm)


def _t2_body(p_ref, dis_ref, b1_ref, w2_ref, g2_ref):
    dis = dis_ref[...]
    h1 = jnp.maximum(dis * (p_ref[0] + p_ref[1]) + b1_ref[...], 0.0)
    g2_ref[...] = jnp.dot(h1, w2_ref[...],
                          preferred_element_type=jnp.float32) * dis


def _t2(p, dis, b1, W2):
    return pl.pallas_call(
        _t2_body,
        grid=(_GRID,),
        in_specs=[
            pl.BlockSpec((NC_, _BR, FD), lambda j: (0, j, 0)),
            pl.BlockSpec((_BR, 1), lambda j: (j, 0)),
            pl.BlockSpec((1, FD), lambda j: (0, 0)),
            pl.BlockSpec((FD, H2_), lambda j: (0, 0)),
        ],
        out_specs=pl.BlockSpec((_BR, H2_), lambda j: (j, 0)),
        out_shape=jax.ShapeDtypeStruct((NN, H2_), jnp.float32),
    )(p, dis, b1, W2)


def _t3_body(q_ref, dis_ref, b2_ref, w0_ref, w1_ref, bo_ref, o0_ref):
    dis = dis_ref[...]
    h2 = jnp.maximum(dis * (q_ref[0] + q_ref[1]) + b2_ref[...], 0.0)
    l0 = jnp.sum(h2 * w0_ref[...], axis=1, keepdims=True) + bo_ref[:, 0:1]
    l1 = jnp.sum(h2 * w1_ref[...], axis=1, keepdims=True) + bo_ref[:, 1:2]
    m = jnp.maximum(l0, l1)
    lse = m + jnp.log(jnp.exp(l0 - m) + jnp.exp(l1 - m))
    o0_ref[...] = jnp.concatenate([l0 - lse, l1 - lse], axis=1)


def _t3(q, dis, b2, w0, w1, bo2):
    return pl.pallas_call(
        _t3_body,
        grid=(_GRID,),
        in_specs=[
            pl.BlockSpec((NC_, _BR, H2_), lambda j: (0, j, 0)),
            pl.BlockSpec((_BR, 1), lambda j: (j, 0)),
            pl.BlockSpec((1, H2_), lambda j: (0, 0)),
            pl.BlockSpec((1, H2_), lambda j: (0, 0)),
            pl.BlockSpec((1, H2_), lambda j: (0, 0)),
            pl.BlockSpec((1, 2), lambda j: (0, 0)),
        ],
        out_specs=pl.BlockSpec((_BR, 2), lambda j: (j, 0)),
        out_shape=jax.ShapeDtypeStruct((NN, 2), jnp.float32),
    )(q, dis, b2, w0, w1, bo2)


def kernel(x, edge_index, W1, b1, W2, b2, Wo, bo):
    mm = _t1a(x, W1)
    degp = _deg_sc(edge_index)
    degT = jnp.transpose(degp[:, :NN])
    g1, dis = _t1b(degT, mm)
    p = _agg128(g1, edge_index)
    g2 = _t2(p, dis, b1.reshape(1, FD), W2)
    q = _agg64(g2, edge_index)
    return _t3(q, dis, b2.reshape(1, H2_),
               Wo[:, 0].reshape(1, H2_), Wo[:, 1].reshape(1, H2_),
               bo.reshape(1, 2))
